# baseline (speedup 1.0000x reference)
import functools

import jax
import jax.numpy as jnp
from jax import lax
from jax.experimental import pallas as pl
from jax.experimental.pallas import tpu as pltpu
from jax.experimental.pallas import tpu_sc as plsc

N, D, K = 4096, 64, 512
BLKN = 1024
NW = 32
RPS = K // NW
HC = N // 2
BIG = 1 << 30


def _tc_dist_body(xt_ref, ct_ref, d2t_ref, out_min_ref):
    xb = xt_ref[...]
    ct = ct_ref[...]
    g = lax.dot_general(
        ct, xb, (((0,), (0,)), ((), ())),
        preferred_element_type=jnp.float32,
        precision=lax.Precision.HIGHEST,
    )
    cn = jnp.sum(ct * ct, axis=0)
    xn = jnp.sum(xb * xb, axis=0)
    d2t = cn[:, None] + xn[None, :] - 2.0 * g
    d2t_ref[...] = d2t
    out_min_ref[...] = jnp.sqrt(jnp.maximum(jnp.min(d2t, axis=0), 0.0))


def _tc_dist(xt, ct):
    return pl.pallas_call(
        _tc_dist_body,
        grid=(N // BLKN,),
        in_specs=[
            pl.BlockSpec((D, BLKN), lambda i: (0, i)),
            pl.BlockSpec((D, K), lambda i: (0, 0)),
        ],
        out_specs=[
            pl.BlockSpec((K, BLKN), lambda i: (0, i)),
            pl.BlockSpec((BLKN,), lambda i: (i,)),
        ],
        out_shape=[
            jax.ShapeDtypeStruct((K, N), jnp.float32),
            jax.ShapeDtypeStruct((N,), jnp.float32),
        ],
        compiler_params=pltpu.CompilerParams(
            dimension_semantics=("parallel",),
        ),
    )(xt, ct)


RH = RPS // 4


def _sc_argmin_body(d2t_hbm, oidx_hbm, buf_v, out_v, sem_a, sem_b, sem_c, sem_d):
    cid = lax.axis_index("c")
    sid = lax.axis_index("s")
    wid = sid * 2 + cid
    r0 = wid * RPS

    sems = (sem_a, sem_b, sem_c, sem_d)
    copies = [
        pltpu.async_copy(
            d2t_hbm.at[pl.ds(r0 + q * RH, RH), :], buf_v.at[q], sems[q]
        )
        for q in range(4)
    ]
    iota16 = lax.iota(jnp.int32, 16)

    def lane_gather(a, perm):
        dn = lax.GatherDimensionNumbers(
            offset_dims=(), collapsed_slice_dims=(0,), start_index_map=(0,))
        return lax.gather(a, perm[:, None], dn, (1,),
                          mode=lax.GatherScatterMode.PROMISE_IN_BOUNDS)

    def row_scan(cur, r):
        def t_loop(t, carry):
            cbv, cbi = carry
            for u in range(16):
                tt = t * 16 + u
                v = buf_v[cur, r, pl.ds(tt * 16, 16)]
                tb = jnp.full((16,), tt, dtype=jnp.int32)
                m = v < cbv
                cbv = jnp.minimum(v, cbv)
                cbi = jnp.where(m, tb, cbi)
            return cbv, cbi

        bv = jnp.full((16,), jnp.inf, dtype=jnp.float32)
        bi = jnp.full((16,), BIG, dtype=jnp.int32)
        bv, bi = lax.fori_loop(0, N // 256, t_loop, (bv, bi))
        bi = bi * 16 + iota16
        for s in (8, 4, 2, 1):
            perm = jnp.bitwise_xor(iota16, s)
            ov = lane_gather(bv, perm)
            oi = lane_gather(bi, perm)
            take = (ov < bv) | ((ov == bv) & (oi < bi))
            bv = jnp.where(take, ov, bv)
            bi = jnp.where(take, oi, bi)
        return bi

    outv = jnp.full((16,), BIG, dtype=jnp.int32)
    for q in range(4):
        copies[q].wait()

        def chunk_rows(r, outv, q=q):
            return jnp.where(iota16 == (r + q * RH), row_scan(q, r), outv)

        outv = lax.fori_loop(0, RH, chunk_rows, outv)
    out_v[...] = outv
    pltpu.sync_copy(out_v, oidx_hbm.at[pl.ds(r0, RPS)])


@functools.partial(
    pl.kernel,
    out_type=jax.ShapeDtypeStruct((K,), jnp.int32),
    mesh=plsc.VectorSubcoreMesh(core_axis_name="c", subcore_axis_name="s"),
    scratch_types=[
        pltpu.VMEM((4, RH, N), jnp.float32),
        pltpu.VMEM((16,), jnp.int32),
        pltpu.SemaphoreType.DMA,
        pltpu.SemaphoreType.DMA,
        pltpu.SemaphoreType.DMA,
        pltpu.SemaphoreType.DMA,
    ],
)
def _sc_argmin(d2t_hbm, oidx_hbm, buf_v, out_v, sem_a, sem_b, sem_c, sem_d):
    _sc_argmin_body(d2t_hbm, oidx_hbm, buf_v, out_v, sem_a, sem_b, sem_c, sem_d)


@jax.jit
def kernel(x, centers):
    d2t, out_min = _tc_dist(x.T, centers.T)
    out_idx = _sc_argmin(d2t)
    return out_idx, out_min, centers


# speedup vs baseline: 1.0068x; 1.0068x over previous
import functools

import jax
import jax.numpy as jnp
from jax import lax
from jax.experimental import pallas as pl
from jax.experimental.pallas import tpu as pltpu
from jax.experimental.pallas import tpu_sc as plsc

N, D, K = 4096, 64, 512
BLKN = 1024
NW = 32
RPS = K // NW
HC = N // 2
BIG = 1 << 30


def _tc_dist_body(xt_ref, ct_ref, d2t_ref, out_min_ref):
    xb = xt_ref[...]
    ct = ct_ref[...]
    g = lax.dot_general(
        ct, xb, (((0,), (0,)), ((), ())),
        preferred_element_type=jnp.float32,
        precision=lax.Precision.HIGHEST,
    )
    cn = jnp.sum(ct * ct, axis=0)
    xn = jnp.sum(xb * xb, axis=0)
    d2t = cn[:, None] + xn[None, :] - 2.0 * g
    d2t_ref[...] = d2t
    out_min_ref[...] = jnp.sqrt(jnp.maximum(jnp.min(d2t, axis=0), 0.0))


def _tc_dist(xt, ct):
    return pl.pallas_call(
        _tc_dist_body,
        grid=(N // BLKN,),
        in_specs=[
            pl.BlockSpec((D, BLKN), lambda i: (0, i)),
            pl.BlockSpec((D, K), lambda i: (0, 0)),
        ],
        out_specs=[
            pl.BlockSpec((K, BLKN), lambda i: (0, i)),
            pl.BlockSpec((BLKN,), lambda i: (i,)),
        ],
        out_shape=[
            jax.ShapeDtypeStruct((K, N), jnp.float32),
            jax.ShapeDtypeStruct((N,), jnp.float32),
        ],
        compiler_params=pltpu.CompilerParams(
            dimension_semantics=("parallel",),
        ),
    )(xt, ct)


RH = RPS // 2


def _sc_argmin_body(d2t_hbm, oidx_hbm, buf_v, out_v, sem_a, sem_b):
    cid = lax.axis_index("c")
    sid = lax.axis_index("s")
    wid = sid * 2 + cid
    r0 = wid * RPS

    sems = (sem_a, sem_b)
    copies = [
        pltpu.async_copy(
            d2t_hbm.at[pl.ds(r0 + q * RH, RH), :], buf_v.at[q], sems[q]
        )
        for q in range(2)
    ]
    iota16 = lax.iota(jnp.int32, 16)

    def lane_gather(a, perm):
        dn = lax.GatherDimensionNumbers(
            offset_dims=(), collapsed_slice_dims=(0,), start_index_map=(0,))
        return lax.gather(a, perm[:, None], dn, (1,),
                          mode=lax.GatherScatterMode.PROMISE_IN_BOUNDS)

    def row_scan(cur, r):
        def t_loop(t, carry):
            cbv, cbi = carry
            for u in range(16):
                tt = t * 16 + u
                v = buf_v[cur, r, pl.ds(tt * 16, 16)]
                tb = jnp.full((16,), tt, dtype=jnp.int32)
                m = v < cbv
                cbv = jnp.minimum(v, cbv)
                cbi = jnp.where(m, tb, cbi)
            return cbv, cbi

        bv = jnp.full((16,), jnp.inf, dtype=jnp.float32)
        bi = jnp.full((16,), BIG, dtype=jnp.int32)
        bv, bi = lax.fori_loop(0, N // 256, t_loop, (bv, bi))
        bi = bi * 16 + iota16
        for s in (8, 4, 2, 1):
            perm = jnp.bitwise_xor(iota16, s)
            ov = lane_gather(bv, perm)
            oi = lane_gather(bi, perm)
            take = (ov < bv) | ((ov == bv) & (oi < bi))
            bv = jnp.where(take, ov, bv)
            bi = jnp.where(take, oi, bi)
        return bi

    outv = jnp.full((16,), BIG, dtype=jnp.int32)
    for q in range(2):
        copies[q].wait()

        def chunk_rows(r, outv, q=q):
            return jnp.where(iota16 == (r + q * RH), row_scan(q, r), outv)

        outv = lax.fori_loop(0, RH, chunk_rows, outv)
    out_v[...] = outv
    pltpu.sync_copy(out_v, oidx_hbm.at[pl.ds(r0, RPS)])


@functools.partial(
    pl.kernel,
    out_type=jax.ShapeDtypeStruct((K,), jnp.int32),
    mesh=plsc.VectorSubcoreMesh(core_axis_name="c", subcore_axis_name="s"),
    scratch_types=[
        pltpu.VMEM((2, RH, N), jnp.float32),
        pltpu.VMEM((16,), jnp.int32),
        pltpu.SemaphoreType.DMA,
        pltpu.SemaphoreType.DMA,
    ],
)
def _sc_argmin(d2t_hbm, oidx_hbm, buf_v, out_v, sem_a, sem_b):
    _sc_argmin_body(d2t_hbm, oidx_hbm, buf_v, out_v, sem_a, sem_b)


@jax.jit
def kernel(x, centers):
    d2t, out_min = _tc_dist(x.T, centers.T)
    out_idx = _sc_argmin(d2t)
    return out_idx, out_min, centers
